# initial kernel scaffold (unmeasured)
import jax
import jax.numpy as jnp
from jax import lax
from jax.experimental import pallas as pl
from jax.experimental.pallas import tpu as pltpu

N_DEV = 4
SQ = 2048
D_MODEL = 1024
HQ = 8
DH = 128
WIN = 128
KV_LOC = 2048
ROWS_B = 128
KV_NEED = KV_LOC + ROWS_B
QBLK = 256
KBLK = 512
NB = SQ // QBLK
CH = SQ // N_DEV
SCALE = 0.08838834764831843
F32 = jnp.float32
MESH = pl.DeviceIdType.MESH


def _flow_a_rdmas(k_ref, v_ref, kr, vr, send_sems, recvA):
    rds = []
    idx = 0
    for j in (1, 2, 3):
        for t, (sref, dref) in enumerate(((k_ref, kr), (v_ref, vr))):
            rds.append(pltpu.make_async_remote_copy(
                src_ref=sref.at[:, HQ * j:HQ * (j + 1), :],
                dst_ref=dref.at[0:KV_LOC],
                send_sem=send_sems.at[idx],
                recv_sem=recvA.at[t],
                device_id=(j,),
                device_id_type=MESH,
            ))
            idx += 1
    return rds


def _flow_b_rdmas(k_ref, v_ref, kr, vr, send_sems, recvB):
    rds = []
    idx = 0
    for j in (0, 2, 3):
        for t, (sref, dref) in enumerate(((k_ref, kr), (v_ref, vr))):
            rds.append(pltpu.make_async_remote_copy(
                src_ref=sref.at[0:ROWS_B, HQ * j:HQ * (j + 1), :],
                dst_ref=dref.at[KV_LOC:KV_NEED],
                send_sem=send_sems.at[idx],
                recv_sem=recvB.at[t],
                device_id=(j,),
                device_id_type=MESH,
            ))
            idx += 1
    return rds


def _local_copies(k_ref, v_ref, kr, vr, loc_sems, my_is_0):
    if my_is_0:
        return [
            pltpu.make_async_copy(k_ref.at[:, 0:HQ, :], kr.at[0:KV_LOC],
                                  loc_sems.at[0]),
            pltpu.make_async_copy(v_ref.at[:, 0:HQ, :], vr.at[0:KV_LOC],
                                  loc_sems.at[1]),
        ]
    return [
        pltpu.make_async_copy(k_ref.at[0:ROWS_B, HQ:2 * HQ, :],
                              kr.at[KV_LOC:KV_NEED], loc_sems.at[0]),
        pltpu.make_async_copy(v_ref.at[0:ROWS_B, HQ:2 * HQ, :],
                              vr.at[KV_LOC:KV_NEED], loc_sems.at[1]),
    ]


def _body(x_ref, wq_ref, k_ref, v_ref, wo_ref, out_ref,
          q_buf, kr, vr, ctx_buf, comm,
          kv_send_sems, loc_sems, recvA, recvB,
          rs_send, rs_recv, ag_send, ag_recv):
    my = lax.axis_index("i")
    right = lax.rem(my + 1, N_DEV)

    bsem = pltpu.get_barrier_semaphore()
    for off in (1, 2, 3):
        pl.semaphore_signal(bsem, inc=1,
                            device_id=(lax.rem(my + off, N_DEV),),
                            device_id_type=MESH)
    pl.semaphore_wait(bsem, N_DEV - 1)

    @pl.when(my == 0)
    def _():
        for rd in _flow_a_rdmas(k_ref, v_ref, kr, vr, kv_send_sems, recvA):
            rd.start()
        for cp in _local_copies(k_ref, v_ref, kr, vr, loc_sems, True):
            cp.start()

    @pl.when(my == 1)
    def _():
        for rd in _flow_b_rdmas(k_ref, v_ref, kr, vr, kv_send_sems, recvB):
            rd.start()
        for cp in _local_copies(k_ref, v_ref, kr, vr, loc_sems, False):
            cp.start()

    q_buf[:, :] = jnp.dot(x_ref[:, :], wq_ref[:, :],
                          preferred_element_type=F32)

    @pl.when(my != 0)
    def _():
        for rd in _flow_a_rdmas(k_ref, v_ref, kr, vr, kv_send_sems, recvA)[:2]:
            rd.wait_recv()

    @pl.when(my != 1)
    def _():
        for rd in _flow_b_rdmas(k_ref, v_ref, kr, vr, kv_send_sems, recvB)[:2]:
            rd.wait_recv()

    @pl.when(my == 0)
    def _():
        for rd in _flow_a_rdmas(k_ref, v_ref, kr, vr, kv_send_sems, recvA):
            rd.wait_send()
        for cp in _local_copies(k_ref, v_ref, kr, vr, loc_sems, True):
            cp.wait()

    @pl.when(my == 1)
    def _():
        for rd in _flow_b_rdmas(k_ref, v_ref, kr, vr, kv_send_sems, recvB):
            rd.wait_send()
        for cp in _local_copies(k_ref, v_ref, kr, vr, loc_sems, False):
            cp.wait()

    for b in range(NB):
        s_b = max(0, QBLK * b - WIN)
        qrows = pl.ds(QBLK * b, QBLK)
        qi = lax.broadcasted_iota(jnp.int32, (QBLK, KBLK), 0) + QBLK * b
        ki = lax.broadcasted_iota(jnp.int32, (QBLK, KBLK), 1) + s_b
        mask = jnp.abs(qi - ki) <= WIN
        for h in range(HQ):
            q = q_buf[qrows, pl.ds(h * DH, DH)]
            k = kr[pl.ds(s_b, KBLK), h, :]
            s = lax.dot_general(q, k, (((1,), (1,)), ((), ())),
                                preferred_element_type=F32) * SCALE
            s = jnp.where(mask, s, -1e9)
            m = jnp.max(s, axis=1, keepdims=True)
            w = jnp.exp(s - m)
            w = w / jnp.sum(w, axis=1, keepdims=True)
            v = vr[pl.ds(s_b, KBLK), h, :]
            ctx_buf[qrows, pl.ds(h * DH, DH)] = jnp.dot(
                w, v, preferred_element_type=F32)

    out_ref[:, :] = jnp.dot(ctx_buf[:, :], wo_ref[:, :],
                            preferred_element_type=F32)

    for h in range(N_DEV - 1):
        sc = lax.rem(my - h + N_DEV, N_DEV)
        rd = pltpu.make_async_remote_copy(
            src_ref=out_ref.at[pl.ds(sc * CH, CH), :],
            dst_ref=comm.at[h],
            send_sem=rs_send.at[h],
            recv_sem=rs_recv.at[h],
            device_id=(right,),
            device_id_type=MESH,
        )
        rd.start()
        rd.wait()
        rc = lax.rem(my - h - 1 + N_DEV, N_DEV)
        rows = pl.ds(rc * CH, CH)
        out_ref[rows, :] = out_ref[rows, :] + comm[h]

    for h in range(N_DEV - 1):
        sc = lax.rem(my + 1 - h + N_DEV, N_DEV)
        rd = pltpu.make_async_remote_copy(
            src_ref=out_ref.at[pl.ds(sc * CH, CH), :],
            dst_ref=out_ref.at[pl.ds(sc * CH, CH), :],
            send_sem=ag_send.at[h],
            recv_sem=ag_recv.at[h],
            device_id=(right,),
            device_id_type=MESH,
        )
        rd.start()
        rd.wait()


def kernel(x, Wq, K_ext, V_ext, Wo):
    x2 = x.reshape(SQ, D_MODEL)
    K2 = K_ext.reshape(KV_LOC, 4 * HQ, DH)
    V2 = V_ext.reshape(KV_LOC, 4 * HQ, DH)

    out2 = pl.pallas_call(
        _body,
        out_shape=jax.ShapeDtypeStruct((SQ, D_MODEL), F32),
        in_specs=[
            pl.BlockSpec(memory_space=pltpu.VMEM),
            pl.BlockSpec(memory_space=pltpu.VMEM),
            pl.BlockSpec(memory_space=pltpu.ANY),
            pl.BlockSpec(memory_space=pltpu.ANY),
            pl.BlockSpec(memory_space=pltpu.VMEM),
        ],
        out_specs=pl.BlockSpec(memory_space=pltpu.VMEM),
        scratch_shapes=[
            pltpu.VMEM((SQ, D_MODEL), F32),
            pltpu.VMEM((KV_NEED, HQ, DH), F32),
            pltpu.VMEM((KV_NEED, HQ, DH), F32),
            pltpu.VMEM((SQ, D_MODEL), F32),
            pltpu.VMEM((N_DEV - 1, CH, D_MODEL), F32),
            pltpu.SemaphoreType.DMA((6,)),
            pltpu.SemaphoreType.DMA((2,)),
            pltpu.SemaphoreType.DMA((2,)),
            pltpu.SemaphoreType.DMA((2,)),
            pltpu.SemaphoreType.DMA((3,)),
            pltpu.SemaphoreType.DMA((3,)),
            pltpu.SemaphoreType.DMA((3,)),
            pltpu.SemaphoreType.DMA((3,)),
        ],
        compiler_params=pltpu.CompilerParams(collective_id=0),
    )(x2, Wq, K2, V2, Wo)
    return out2.reshape(1, SQ, D_MODEL)


# baseline (device time: 580715 ns/iter reference)
import jax
import jax.numpy as jnp
from jax import lax
from jax.experimental import pallas as pl
from jax.experimental.pallas import tpu as pltpu

N_DEV = 4
SQ = 2048
D_MODEL = 1024
HQ = 8
DH = 128
WIN = 128
KV_LOC = 2048
ROWS_B = 128
KV_NEED = KV_LOC + ROWS_B
QBLK = 256
KBLK = 512
NB = SQ // QBLK
CH = SQ // N_DEV
SCALE = 0.08838834764831843
F32 = jnp.float32
MESH = pl.DeviceIdType.MESH


def _flow_a_rdmas(k_ref, v_ref, kr, vr, send_sems, recvA):
    rds = []
    idx = 0
    for j in (1, 2, 3):
        for t, (sref, dref) in enumerate(((k_ref, kr), (v_ref, vr))):
            rds.append(pltpu.make_async_remote_copy(
                src_ref=sref.at[:, HQ * j:HQ * (j + 1), :],
                dst_ref=dref.at[0:KV_LOC],
                send_sem=send_sems.at[idx],
                recv_sem=recvA.at[t],
                device_id=(j,),
                device_id_type=MESH,
            ))
            idx += 1
    return rds


def _flow_b_rdmas(k_ref, v_ref, kr, vr, send_sems, recvB):
    rds = []
    idx = 0
    for j in (0, 2, 3):
        for t, (sref, dref) in enumerate(((k_ref, kr), (v_ref, vr))):
            rds.append(pltpu.make_async_remote_copy(
                src_ref=sref.at[0:ROWS_B, HQ * j:HQ * (j + 1), :],
                dst_ref=dref.at[KV_LOC:KV_NEED],
                send_sem=send_sems.at[idx],
                recv_sem=recvB.at[t],
                device_id=(j,),
                device_id_type=MESH,
            ))
            idx += 1
    return rds


def _local_copies(k_ref, v_ref, kr, vr, loc_sems, my_is_0):
    if my_is_0:
        return [
            pltpu.make_async_copy(k_ref.at[:, 0:HQ, :], kr.at[0:KV_LOC],
                                  loc_sems.at[0]),
            pltpu.make_async_copy(v_ref.at[:, 0:HQ, :], vr.at[0:KV_LOC],
                                  loc_sems.at[1]),
        ]
    return [
        pltpu.make_async_copy(k_ref.at[0:ROWS_B, HQ:2 * HQ, :],
                              kr.at[KV_LOC:KV_NEED], loc_sems.at[0]),
        pltpu.make_async_copy(v_ref.at[0:ROWS_B, HQ:2 * HQ, :],
                              vr.at[KV_LOC:KV_NEED], loc_sems.at[1]),
    ]


def _body(x_ref, wq_ref, k_ref, v_ref, wo_ref, out_ref,
          kr, vr, comm,
          kv_send_sems, loc_sems, recvA, recvB,
          rs_send, rs_recv, ag_send, ag_recv):
    my = lax.axis_index("i")
    right = lax.rem(my + 1, N_DEV)

    bsem = pltpu.get_barrier_semaphore()
    for off in (1, 2, 3):
        pl.semaphore_signal(bsem, inc=1,
                            device_id=(lax.rem(my + off, N_DEV),),
                            device_id_type=MESH)
    pl.semaphore_wait(bsem, N_DEV - 1)

    @pl.when(my == 0)
    def _():
        for rd in _flow_a_rdmas(k_ref, v_ref, kr, vr, kv_send_sems, recvA):
            rd.start()
        for cp in _local_copies(k_ref, v_ref, kr, vr, loc_sems, True):
            cp.start()

    @pl.when(my == 1)
    def _():
        for rd in _flow_b_rdmas(k_ref, v_ref, kr, vr, kv_send_sems, recvB):
            rd.start()
        for cp in _local_copies(k_ref, v_ref, kr, vr, loc_sems, False):
            cp.start()

    @pl.when(my != 0)
    def _():
        for rd in _flow_a_rdmas(k_ref, v_ref, kr, vr, kv_send_sems, recvA)[:2]:
            rd.wait_recv()

    @pl.when(my != 1)
    def _():
        for rd in _flow_b_rdmas(k_ref, v_ref, kr, vr, kv_send_sems, recvB)[:2]:
            rd.wait_recv()

    @pl.when(my == 0)
    def _():
        for rd in _flow_a_rdmas(k_ref, v_ref, kr, vr, kv_send_sems, recvA):
            rd.wait_send()
        for cp in _local_copies(k_ref, v_ref, kr, vr, loc_sems, True):
            cp.wait()

    @pl.when(my == 1)
    def _():
        for rd in _flow_b_rdmas(k_ref, v_ref, kr, vr, kv_send_sems, recvB):
            rd.wait_send()
        for cp in _local_copies(k_ref, v_ref, kr, vr, loc_sems, False):
            cp.wait()

    for b in range(NB):
        s_b = max(0, QBLK * b - WIN)
        qrows = pl.ds(QBLK * b, QBLK)
        qi = lax.broadcasted_iota(jnp.int32, (QBLK, KBLK), 0) + QBLK * b
        ki = lax.broadcasted_iota(jnp.int32, (QBLK, KBLK), 1) + s_b
        mask = jnp.abs(qi - ki) <= WIN
        q_blk = jnp.dot(x_ref[qrows, :], wq_ref[:, :],
                        preferred_element_type=F32)
        acc = jnp.zeros((QBLK, D_MODEL), F32)
        for h in range(HQ):
            q = q_blk[:, h * DH:(h + 1) * DH]
            k = kr[pl.ds(s_b, KBLK), h, :]
            s = lax.dot_general(q, k, (((1,), (1,)), ((), ())),
                                preferred_element_type=F32) * SCALE
            s = jnp.where(mask, s, -1e9)
            m = jnp.max(s, axis=1, keepdims=True)
            w = jnp.exp(s - m)
            w = w / jnp.sum(w, axis=1, keepdims=True)
            v = vr[pl.ds(s_b, KBLK), h, :]
            ctx = jnp.dot(w, v, preferred_element_type=F32)
            acc = acc + jnp.dot(ctx, wo_ref[pl.ds(h * DH, DH), :],
                                preferred_element_type=F32)
        out_ref[qrows, :] = acc

    for h in range(N_DEV - 1):
        sc = lax.rem(my - h + N_DEV, N_DEV)
        rd = pltpu.make_async_remote_copy(
            src_ref=out_ref.at[pl.ds(sc * CH, CH), :],
            dst_ref=comm.at[h],
            send_sem=rs_send.at[h],
            recv_sem=rs_recv.at[h],
            device_id=(right,),
            device_id_type=MESH,
        )
        rd.start()
        rd.wait()
        rc = lax.rem(my - h - 1 + N_DEV, N_DEV)
        rows = pl.ds(rc * CH, CH)
        out_ref[rows, :] = out_ref[rows, :] + comm[h]

    for h in range(N_DEV - 1):
        sc = lax.rem(my + 1 - h + N_DEV, N_DEV)
        rd = pltpu.make_async_remote_copy(
            src_ref=out_ref.at[pl.ds(sc * CH, CH), :],
            dst_ref=out_ref.at[pl.ds(sc * CH, CH), :],
            send_sem=ag_send.at[h],
            recv_sem=ag_recv.at[h],
            device_id=(right,),
            device_id_type=MESH,
        )
        rd.start()
        rd.wait()


def kernel(x, Wq, K_ext, V_ext, Wo):
    x2 = x.reshape(SQ, D_MODEL)
    K2 = K_ext.reshape(KV_LOC, 4 * HQ, DH)
    V2 = V_ext.reshape(KV_LOC, 4 * HQ, DH)

    out2 = pl.pallas_call(
        _body,
        out_shape=jax.ShapeDtypeStruct((SQ, D_MODEL), F32),
        in_specs=[
            pl.BlockSpec(memory_space=pltpu.VMEM),
            pl.BlockSpec(memory_space=pltpu.VMEM),
            pl.BlockSpec(memory_space=pl.ANY),
            pl.BlockSpec(memory_space=pl.ANY),
            pl.BlockSpec(memory_space=pltpu.VMEM),
        ],
        out_specs=pl.BlockSpec(memory_space=pltpu.VMEM),
        scratch_shapes=[
            pltpu.VMEM((KV_NEED, HQ, DH), F32),
            pltpu.VMEM((KV_NEED, HQ, DH), F32),
            pltpu.VMEM((N_DEV - 1, CH, D_MODEL), F32),
            pltpu.SemaphoreType.DMA((6,)),
            pltpu.SemaphoreType.DMA((2,)),
            pltpu.SemaphoreType.DMA((2,)),
            pltpu.SemaphoreType.DMA((2,)),
            pltpu.SemaphoreType.DMA((3,)),
            pltpu.SemaphoreType.DMA((3,)),
            pltpu.SemaphoreType.DMA((3,)),
            pltpu.SemaphoreType.DMA((3,)),
        ],
        compiler_params=pltpu.CompilerParams(
            collective_id=0,
            vmem_limit_bytes=128 * 1024 * 1024,
        ),
    )(x2, Wq, K2, V2, Wo)
    return out2.reshape(1, SQ, D_MODEL)


# device time: 445966 ns/iter; 1.3022x vs baseline; 1.3022x over previous
import jax
import jax.numpy as jnp
from jax import lax
from jax.experimental import pallas as pl
from jax.experimental.pallas import tpu as pltpu

N_DEV = 4
SQ = 2048
D_MODEL = 1024
HQ = 8
DH = 128
WIN = 128
KV_LOC = 2048
ROWS_B = 128
KV_NEED = KV_LOC + ROWS_B
QBLK = 256
KBLK = 512
NB = SQ // QBLK
CH = SQ // N_DEV
SCALE = 0.08838834764831843
F32 = jnp.float32
BF16 = jnp.bfloat16
MESH = pl.DeviceIdType.MESH


def _flow_a_rdmas(k_ref, v_ref, kr, vr, send_sems, recvA, dests=(1, 2, 3)):
    rds = []
    for j in dests:
        for t, (sref, dref) in enumerate(((k_ref, kr), (v_ref, vr))):
            rds.append(pltpu.make_async_remote_copy(
                src_ref=sref.at[:, HQ * j:HQ * (j + 1), :],
                dst_ref=dref.at[0:KV_LOC],
                send_sem=send_sems.at[2 * (j - 1) + t],
                recv_sem=recvA.at[t],
                device_id=(j,),
                device_id_type=MESH,
            ))
    return rds


def _flow_b_rdmas(k_ref, v_ref, kr, vr, send_sems, recvB, dests=(0, 2, 3)):
    rds = []
    for i, j in enumerate(dests):
        for t, (sref, dref) in enumerate(((k_ref, kr), (v_ref, vr))):
            rds.append(pltpu.make_async_remote_copy(
                src_ref=sref.at[0:ROWS_B, HQ * j:HQ * (j + 1), :],
                dst_ref=dref.at[KV_LOC:KV_NEED],
                send_sem=send_sems.at[2 * i + t],
                recv_sem=recvB.at[t],
                device_id=(j,),
                device_id_type=MESH,
            ))
    return rds


def _local_copies(k_ref, v_ref, kr, vr, loc_sems, my_is_0):
    if my_is_0:
        return [
            pltpu.make_async_copy(k_ref.at[:, 0:HQ, :], kr.at[0:KV_LOC],
                                  loc_sems.at[0]),
            pltpu.make_async_copy(v_ref.at[:, 0:HQ, :], vr.at[0:KV_LOC],
                                  loc_sems.at[1]),
        ]
    return [
        pltpu.make_async_copy(k_ref.at[0:ROWS_B, HQ:2 * HQ, :],
                              kr.at[KV_LOC:KV_NEED], loc_sems.at[0]),
        pltpu.make_async_copy(v_ref.at[0:ROWS_B, HQ:2 * HQ, :],
                              vr.at[KV_LOC:KV_NEED], loc_sems.at[1]),
    ]


def _body(x_ref, wq_ref, k_ref, v_ref, wo_ref, out_ref,
          q_buf, kr, vr, comm,
          kv_send_sems, loc_sems, recvA, recvB,
          rs_send, rs_recv, ag_send, ag_recv):
    my = lax.axis_index("i")
    right = lax.rem(my + 1, N_DEV)

    bsem = pltpu.get_barrier_semaphore()
    for off in (1, 2, 3):
        pl.semaphore_signal(bsem, inc=1,
                            device_id=(lax.rem(my + off, N_DEV),),
                            device_id_type=MESH)
    pl.semaphore_wait(bsem, N_DEV - 1)

    @pl.when(my == 0)
    def _():
        for rd in _flow_a_rdmas(k_ref, v_ref, kr, vr, kv_send_sems, recvA):
            rd.start()
        for cp in _local_copies(k_ref, v_ref, kr, vr, loc_sems, True):
            cp.start()

    @pl.when(my == 1)
    def _():
        for rd in _flow_b_rdmas(k_ref, v_ref, kr, vr, kv_send_sems, recvB):
            rd.start()
        for cp in _local_copies(k_ref, v_ref, kr, vr, loc_sems, False):
            cp.start()

    q_buf[:, :] = jnp.dot(x_ref[:, :], wq_ref[:, :],
                          preferred_element_type=F32)

    @pl.when(my != 0)
    def _():
        for rd in _flow_a_rdmas(k_ref, v_ref, kr, vr, kv_send_sems, recvA,
                                dests=(1,)):
            rd.wait_recv()

    @pl.when(my != 1)
    def _():
        for rd in _flow_b_rdmas(k_ref, v_ref, kr, vr, kv_send_sems, recvB,
                                dests=(0,)):
            rd.wait_recv()

    @pl.when(my == 0)
    def _():
        for rd in _flow_a_rdmas(k_ref, v_ref, kr, vr, kv_send_sems, recvA):
            rd.wait_send()
        for cp in _local_copies(k_ref, v_ref, kr, vr, loc_sems, True):
            cp.wait()

    @pl.when(my == 1)
    def _():
        for rd in _flow_b_rdmas(k_ref, v_ref, kr, vr, kv_send_sems, recvB):
            rd.wait_send()
        for cp in _local_copies(k_ref, v_ref, kr, vr, loc_sems, False):
            cp.wait()

    for b in range(NB):
        s_b = max(0, QBLK * b - WIN)
        qrows = pl.ds(QBLK * b, QBLK)
        qi = lax.broadcasted_iota(jnp.int32, (QBLK, KBLK), 0) + QBLK * b
        ki = lax.broadcasted_iota(jnp.int32, (QBLK, KBLK), 1) + s_b
        mask = jnp.abs(qi - ki) <= WIN
        acc = jnp.zeros((QBLK, D_MODEL), F32)
        for h in range(HQ):
            q = q_buf[qrows, pl.ds(h * DH, DH)].astype(BF16)
            k = kr[pl.ds(s_b, KBLK), h, :]
            s = lax.dot_general(q, k, (((1,), (1,)), ((), ())),
                                preferred_element_type=F32) * SCALE
            s = jnp.where(mask, s, -1e9)
            m = jnp.max(s, axis=1, keepdims=True)
            w = jnp.exp(s - m)
            w = (w / jnp.sum(w, axis=1, keepdims=True)).astype(BF16)
            v = vr[pl.ds(s_b, KBLK), h, :]
            ctx = jnp.dot(w, v, preferred_element_type=F32)
            acc = acc + jnp.dot(ctx, wo_ref[pl.ds(h * DH, DH), :],
                                preferred_element_type=F32)
        out_ref[qrows, :] = acc

    for h in range(N_DEV - 1):
        sc = lax.rem(my - h + N_DEV, N_DEV)
        rd = pltpu.make_async_remote_copy(
            src_ref=out_ref.at[pl.ds(sc * CH, CH), :],
            dst_ref=comm.at[h],
            send_sem=rs_send.at[h],
            recv_sem=rs_recv.at[h],
            device_id=(right,),
            device_id_type=MESH,
        )
        rd.start()
        rd.wait()
        rc = lax.rem(my - h - 1 + N_DEV, N_DEV)
        rows = pl.ds(rc * CH, CH)
        out_ref[rows, :] = out_ref[rows, :] + comm[h]

    for h in range(N_DEV - 1):
        sc = lax.rem(my + 1 - h + N_DEV, N_DEV)
        rd = pltpu.make_async_remote_copy(
            src_ref=out_ref.at[pl.ds(sc * CH, CH), :],
            dst_ref=out_ref.at[pl.ds(sc * CH, CH), :],
            send_sem=ag_send.at[h],
            recv_sem=ag_recv.at[h],
            device_id=(right,),
            device_id_type=MESH,
        )
        rd.start()
        rd.wait()


def kernel(x, Wq, K_ext, V_ext, Wo):
    x2 = x.reshape(SQ, D_MODEL)
    Kb = K_ext.reshape(KV_LOC, 4 * HQ, DH).astype(BF16)
    Vb = V_ext.reshape(KV_LOC, 4 * HQ, DH).astype(BF16)

    out2 = pl.pallas_call(
        _body,
        out_shape=jax.ShapeDtypeStruct((SQ, D_MODEL), F32),
        in_specs=[
            pl.BlockSpec(memory_space=pltpu.VMEM),
            pl.BlockSpec(memory_space=pltpu.VMEM),
            pl.BlockSpec(memory_space=pl.ANY),
            pl.BlockSpec(memory_space=pl.ANY),
            pl.BlockSpec(memory_space=pltpu.VMEM),
        ],
        out_specs=pl.BlockSpec(memory_space=pltpu.VMEM),
        scratch_shapes=[
            pltpu.VMEM((SQ, D_MODEL), F32),
            pltpu.VMEM((KV_NEED, HQ, DH), BF16),
            pltpu.VMEM((KV_NEED, HQ, DH), BF16),
            pltpu.VMEM((N_DEV - 1, CH, D_MODEL), F32),
            pltpu.SemaphoreType.DMA((6,)),
            pltpu.SemaphoreType.DMA((2,)),
            pltpu.SemaphoreType.DMA((2,)),
            pltpu.SemaphoreType.DMA((2,)),
            pltpu.SemaphoreType.DMA((3,)),
            pltpu.SemaphoreType.DMA((3,)),
            pltpu.SemaphoreType.DMA((3,)),
            pltpu.SemaphoreType.DMA((3,)),
        ],
        compiler_params=pltpu.CompilerParams(
            collective_id=0,
            vmem_limit_bytes=128 * 1024 * 1024,
        ),
    )(x2, Wq, Kb, Vb, Wo)
    return out2.reshape(1, SQ, D_MODEL)


# device time: 308296 ns/iter; 1.8836x vs baseline; 1.4466x over previous
import jax
import jax.numpy as jnp
from jax import lax
from jax.experimental import pallas as pl
from jax.experimental.pallas import tpu as pltpu

N_DEV = 4
SQ = 2048
D_MODEL = 1024
HQ = 8
DH = 128
WIN = 128
KV_LOC = 2048
ROWS_B = 128
KV_NEED = KV_LOC + ROWS_B
QBLK = 256
KBLK = 512
NB = SQ // QBLK
CHUNK = 512
NCH = KV_LOC // CHUNK
CH = SQ // N_DEV
SCALE = 0.08838834764831843
F32 = jnp.float32
BF16 = jnp.bfloat16
MESH = pl.DeviceIdType.MESH


def _flow_a_rdma(k_ref, v_ref, kr, vr, send_sems, recvA, j, c, t):
    sref, dref = ((k_ref, kr), (v_ref, vr))[t]
    return pltpu.make_async_remote_copy(
        src_ref=sref.at[CHUNK * c:CHUNK * (c + 1), HQ * j:HQ * (j + 1), :],
        dst_ref=dref.at[CHUNK * c:CHUNK * (c + 1)],
        send_sem=send_sems.at[8 * (j - 1) + 2 * c + t],
        recv_sem=recvA.at[2 * c + t],
        device_id=(j,),
        device_id_type=MESH,
    )


def _flow_b_rdma(k_ref, v_ref, kr, vr, send_sems, recvB, i, j, t):
    sref, dref = ((k_ref, kr), (v_ref, vr))[t]
    return pltpu.make_async_remote_copy(
        src_ref=sref.at[0:ROWS_B, HQ * j:HQ * (j + 1), :],
        dst_ref=dref.at[KV_LOC:KV_NEED],
        send_sem=send_sems.at[2 * i + t],
        recv_sem=recvB.at[t],
        device_id=(j,),
        device_id_type=MESH,
    )


def _local_copies(k_ref, v_ref, kr, vr, loc_sems, my_is_0):
    if my_is_0:
        return [
            pltpu.make_async_copy(k_ref.at[:, 0:HQ, :], kr.at[0:KV_LOC],
                                  loc_sems.at[0]),
            pltpu.make_async_copy(v_ref.at[:, 0:HQ, :], vr.at[0:KV_LOC],
                                  loc_sems.at[1]),
        ]
    return [
        pltpu.make_async_copy(k_ref.at[0:ROWS_B, HQ:2 * HQ, :],
                              kr.at[KV_LOC:KV_NEED], loc_sems.at[0]),
        pltpu.make_async_copy(v_ref.at[0:ROWS_B, HQ:2 * HQ, :],
                              vr.at[KV_LOC:KV_NEED], loc_sems.at[1]),
    ]


def _body(x_ref, wq_ref, k_ref, v_ref, wo_ref, out_ref,
          q_buf, kr, vr, ar_buf, rs_slots,
          kv_send_sems, fb_send_sems, loc_sems, recvA, recvB,
          rs_send, rs_recv, ag_send, ag_recv):
    my = lax.axis_index("i")

    bsem = pltpu.get_barrier_semaphore()
    for off in (1, 2, 3):
        pl.semaphore_signal(bsem, inc=1,
                            device_id=(lax.rem(my + off, N_DEV),),
                            device_id_type=MESH)
    pl.semaphore_wait(bsem, N_DEV - 1)

    @pl.when(my == 0)
    def _():
        for c in range(NCH):
            for j in (1, 2, 3):
                for t in (0, 1):
                    _flow_a_rdma(k_ref, v_ref, kr, vr,
                                 kv_send_sems, recvA, j, c, t).start()
        for cp in _local_copies(k_ref, v_ref, kr, vr, loc_sems, True):
            cp.start()

    @pl.when(my == 1)
    def _():
        for i, j in enumerate((0, 2, 3)):
            for t in (0, 1):
                _flow_b_rdma(k_ref, v_ref, kr, vr,
                             fb_send_sems, recvB, i, j, t).start()
        for cp in _local_copies(k_ref, v_ref, kr, vr, loc_sems, False):
            cp.start()

    q_buf[:, :] = jnp.dot(x_ref[:, :], wq_ref[:, :],
                          preferred_element_type=F32)

    @pl.when(my == 0)
    def _():
        for cp in _local_copies(k_ref, v_ref, kr, vr, loc_sems, True):
            cp.wait()

    @pl.when(my == 1)
    def _():
        for cp in _local_copies(k_ref, v_ref, kr, vr, loc_sems, False):
            cp.wait()

    waited = 0
    for b in range(NB):
        hi = min((QBLK * b + QBLK + WIN - 1) // CHUNK, NCH - 1)
        for c in range(waited, hi + 1):
            @pl.when(my != 0)
            def _(c=c):
                for t in (0, 1):
                    _flow_a_rdma(k_ref, v_ref, kr, vr,
                                 kv_send_sems, recvA, 1, c, t).wait_recv()
        waited = max(waited, hi + 1)
        if b == NB - 1:
            @pl.when(my != 1)
            def _():
                for t in (0, 1):
                    _flow_b_rdma(k_ref, v_ref, kr, vr,
                                 fb_send_sems, recvB, 0, 0, t).wait_recv()

        s_b = max(0, QBLK * b - WIN)
        qrows = pl.ds(QBLK * b, QBLK)
        qi = lax.broadcasted_iota(jnp.int32, (QBLK, KBLK), 0) + QBLK * b
        ki = lax.broadcasted_iota(jnp.int32, (QBLK, KBLK), 1) + s_b
        mask = jnp.abs(qi - ki) <= WIN
        acc = jnp.zeros((QBLK, D_MODEL), F32)
        for h in range(HQ):
            q = q_buf[qrows, pl.ds(h * DH, DH)].astype(BF16)
            k = kr[pl.ds(s_b, KBLK), h, :]
            s = lax.dot_general(q, k, (((1,), (1,)), ((), ())),
                                preferred_element_type=F32) * SCALE
            s = jnp.where(mask, s, -1e9)
            m = jnp.max(s, axis=1, keepdims=True)
            w = jnp.exp(s - m)
            w = (w / jnp.sum(w, axis=1, keepdims=True)).astype(BF16)
            v = vr[pl.ds(s_b, KBLK), h, :]
            ctx = jnp.dot(w, v, preferred_element_type=F32)
            acc = acc + jnp.dot(ctx, wo_ref[pl.ds(h * DH, DH), :],
                                preferred_element_type=F32)
        ar_buf[qrows, :] = acc.astype(BF16)

    @pl.when(my == 0)
    def _():
        for c in range(NCH):
            for j in (1, 2, 3):
                for t in (0, 1):
                    _flow_a_rdma(k_ref, v_ref, kr, vr,
                                 kv_send_sems, recvA, j, c, t).wait_send()

    @pl.when(my == 1)
    def _():
        for i, j in enumerate((0, 2, 3)):
            for t in (0, 1):
                _flow_b_rdma(k_ref, v_ref, kr, vr,
                             fb_send_sems, recvB, i, j, t).wait_send()

    for off in (1, 2, 3):
        dest = lax.rem(my + off, N_DEV)
        pltpu.make_async_remote_copy(
            src_ref=ar_buf.at[pl.ds(dest * CH, CH), :],
            dst_ref=rs_slots.at[3 - off],
            send_sem=rs_send.at[off - 1],
            recv_sem=rs_recv.at[3 - off],
            device_id=(dest,),
            device_id_type=MESH,
        ).start()
    for slot in range(3):
        pltpu.make_async_remote_copy(
            src_ref=ar_buf.at[pl.ds(0, CH), :],
            dst_ref=rs_slots.at[slot],
            send_sem=rs_send.at[0],
            recv_sem=rs_recv.at[slot],
            device_id=(0,),
            device_id_type=MESH,
        ).wait_recv()
    myrows = pl.ds(my * CH, CH)
    red = (ar_buf[myrows, :].astype(F32)
           + rs_slots[0].astype(F32)
           + rs_slots[1].astype(F32)
           + rs_slots[2].astype(F32))
    ar_buf[myrows, :] = red.astype(BF16)

    for off in (1, 2, 3):
        dest = lax.rem(my + off, N_DEV)
        pltpu.make_async_remote_copy(
            src_ref=ar_buf.at[myrows, :],
            dst_ref=ar_buf.at[myrows, :],
            send_sem=ag_send.at[off - 1],
            recv_sem=ag_recv.at[3 - off],
            device_id=(dest,),
            device_id_type=MESH,
        ).start()
    for slot in range(3):
        pltpu.make_async_remote_copy(
            src_ref=ar_buf.at[pl.ds(0, CH), :],
            dst_ref=ar_buf.at[pl.ds(0, CH), :],
            send_sem=ag_send.at[0],
            recv_sem=ag_recv.at[slot],
            device_id=(0,),
            device_id_type=MESH,
        ).wait_recv()
    for off in (1, 2, 3):
        pltpu.make_async_remote_copy(
            src_ref=ar_buf.at[pl.ds(0, CH), :],
            dst_ref=rs_slots.at[0],
            send_sem=rs_send.at[off - 1],
            recv_sem=rs_recv.at[0],
            device_id=(0,),
            device_id_type=MESH,
        ).wait_send()
        pltpu.make_async_remote_copy(
            src_ref=ar_buf.at[pl.ds(0, CH), :],
            dst_ref=rs_slots.at[0],
            send_sem=ag_send.at[off - 1],
            recv_sem=rs_recv.at[0],
            device_id=(0,),
            device_id_type=MESH,
        ).wait_send()

    out_ref[:, :] = ar_buf[:, :].astype(F32)


def kernel(x, Wq, K_ext, V_ext, Wo):
    x2 = x.reshape(SQ, D_MODEL)
    Kb = K_ext.reshape(KV_LOC, 4 * HQ, DH).astype(BF16)
    Vb = V_ext.reshape(KV_LOC, 4 * HQ, DH).astype(BF16)

    out2 = pl.pallas_call(
        _body,
        out_shape=jax.ShapeDtypeStruct((SQ, D_MODEL), F32),
        in_specs=[
            pl.BlockSpec(memory_space=pltpu.VMEM),
            pl.BlockSpec(memory_space=pltpu.VMEM),
            pl.BlockSpec(memory_space=pl.ANY),
            pl.BlockSpec(memory_space=pl.ANY),
            pl.BlockSpec(memory_space=pltpu.VMEM),
        ],
        out_specs=pl.BlockSpec(memory_space=pltpu.VMEM),
        scratch_shapes=[
            pltpu.VMEM((SQ, D_MODEL), F32),
            pltpu.VMEM((KV_NEED, HQ, DH), BF16),
            pltpu.VMEM((KV_NEED, HQ, DH), BF16),
            pltpu.VMEM((SQ, D_MODEL), BF16),
            pltpu.VMEM((3, CH, D_MODEL), BF16),
            pltpu.SemaphoreType.DMA((8 * 3,)),
            pltpu.SemaphoreType.DMA((6,)),
            pltpu.SemaphoreType.DMA((2,)),
            pltpu.SemaphoreType.DMA((2 * NCH,)),
            pltpu.SemaphoreType.DMA((2,)),
            pltpu.SemaphoreType.DMA((3,)),
            pltpu.SemaphoreType.DMA((3,)),
            pltpu.SemaphoreType.DMA((3,)),
            pltpu.SemaphoreType.DMA((3,)),
        ],
        compiler_params=pltpu.CompilerParams(
            collective_id=0,
            vmem_limit_bytes=128 * 1024 * 1024,
        ),
    )(x2, Wq, Kb, Vb, Wo)
    return out2.reshape(1, SQ, D_MODEL)


# device time: 287176 ns/iter; 2.0222x vs baseline; 1.0735x over previous
import jax
import jax.numpy as jnp
from jax import lax
from jax.experimental import pallas as pl
from jax.experimental.pallas import tpu as pltpu

N_DEV = 4
SQ = 2048
D_MODEL = 1024
HQ = 8
DH = 128
WIN = 128
KV_LOC = 2048
ROWS_B = 128
KV_NEED = KV_LOC + ROWS_B
QBLK = 256
KBLK = 512
NB = SQ // QBLK
CHUNK = 512
NCH = KV_LOC // CHUNK
CH = SQ // N_DEV
SCALE = 0.08838834764831843
F32 = jnp.float32
BF16 = jnp.bfloat16
MESH = pl.DeviceIdType.MESH


def _flow_a_direct(k_ref, v_ref, kr, vr, send_sems, recvA, j, c, t):
    sref, dref = ((k_ref, kr), (v_ref, vr))[t]
    di = 0 if j == 1 else 1
    return pltpu.make_async_remote_copy(
        src_ref=sref.at[CHUNK * c:CHUNK * (c + 1), HQ * j:HQ * (j + 1), :],
        dst_ref=dref.at[CHUNK * c:CHUNK * (c + 1)],
        send_sem=send_sems.at[8 * di + 2 * c + t],
        recv_sem=recvA.at[2 * c + t],
        device_id=(j,),
        device_id_type=MESH,
    )


def _flow_a_relay(k_ref, v_ref, relay_buf, send_sems, relay_recv, c, t):
    sref = (k_ref, v_ref)[t]
    return pltpu.make_async_remote_copy(
        src_ref=sref.at[CHUNK * c:CHUNK * (c + 1), 2 * HQ:3 * HQ, :],
        dst_ref=relay_buf.at[CHUNK * c:CHUNK * (c + 1)],
        send_sem=send_sems.at[16 + 2 * c + t],
        recv_sem=relay_recv.at[c],
        device_id=(1 if t == 0 else 3,),
        device_id_type=MESH,
    )


def _relay_fwd(relay_buf, kr, vr, fwd_send, recvA, c, t):
    dref = (kr, vr)[t]
    return pltpu.make_async_remote_copy(
        src_ref=relay_buf.at[CHUNK * c:CHUNK * (c + 1)],
        dst_ref=dref.at[CHUNK * c:CHUNK * (c + 1)],
        send_sem=fwd_send.at[c],
        recv_sem=recvA.at[2 * c + t],
        device_id=(2,),
        device_id_type=MESH,
    )


def _flow_b_rdma(k_ref, v_ref, kr, vr, send_sems, recvB, i, j, t):
    sref, dref = ((k_ref, kr), (v_ref, vr))[t]
    return pltpu.make_async_remote_copy(
        src_ref=sref.at[0:ROWS_B, HQ * j:HQ * (j + 1), :],
        dst_ref=dref.at[KV_LOC:KV_NEED],
        send_sem=send_sems.at[2 * i + t],
        recv_sem=recvB.at[t],
        device_id=(j,),
        device_id_type=MESH,
    )


def _local_copies(k_ref, v_ref, kr, vr, loc_sems, my_is_0):
    if my_is_0:
        return [
            pltpu.make_async_copy(k_ref.at[:, 0:HQ, :], kr.at[0:KV_LOC],
                                  loc_sems.at[0]),
            pltpu.make_async_copy(v_ref.at[:, 0:HQ, :], vr.at[0:KV_LOC],
                                  loc_sems.at[1]),
        ]
    return [
        pltpu.make_async_copy(k_ref.at[0:ROWS_B, HQ:2 * HQ, :],
                              kr.at[KV_LOC:KV_NEED], loc_sems.at[0]),
        pltpu.make_async_copy(v_ref.at[0:ROWS_B, HQ:2 * HQ, :],
                              vr.at[KV_LOC:KV_NEED], loc_sems.at[1]),
    ]


def _body(x_ref, wq_ref, k_ref, v_ref, wo_ref, out_ref,
          q_buf, kr, vr, ar_buf, rs_slots, relay_buf,
          kv_send_sems, fb_send_sems, loc_sems, fwd_send, relay_recv,
          recvA, recvB, rs_send, rs_recv, ag_send, ag_recv):
    my = lax.axis_index("i")

    bsem = pltpu.get_barrier_semaphore()
    for off in (1, 2, 3):
        pl.semaphore_signal(bsem, inc=1,
                            device_id=(lax.rem(my + off, N_DEV),),
                            device_id_type=MESH)
    pl.semaphore_wait(bsem, N_DEV - 1)

    @pl.when(my == 0)
    def _():
        for c in range(NCH):
            for t in (0, 1):
                _flow_a_direct(k_ref, v_ref, kr, vr,
                               kv_send_sems, recvA, 1, c, t).start()
                _flow_a_direct(k_ref, v_ref, kr, vr,
                               kv_send_sems, recvA, 3, c, t).start()
                _flow_a_relay(k_ref, v_ref, relay_buf,
                              kv_send_sems, relay_recv, c, t).start()
        for cp in _local_copies(k_ref, v_ref, kr, vr, loc_sems, True):
            cp.start()

    @pl.when(my == 1)
    def _():
        for i, j in enumerate((0, 2, 3)):
            for t in (0, 1):
                _flow_b_rdma(k_ref, v_ref, kr, vr,
                             fb_send_sems, recvB, i, j, t).start()
        for cp in _local_copies(k_ref, v_ref, kr, vr, loc_sems, False):
            cp.start()

    q_buf[:, :] = jnp.dot(x_ref[:, :], wq_ref[:, :],
                          preferred_element_type=F32).astype(BF16)

    @pl.when(my == 0)
    def _():
        for cp in _local_copies(k_ref, v_ref, kr, vr, loc_sems, True):
            cp.wait()

    @pl.when(my == 1)
    def _():
        for cp in _local_copies(k_ref, v_ref, kr, vr, loc_sems, False):
            cp.wait()

    waited = 0
    for b in range(NB):
        hi = min((QBLK * b + QBLK + WIN - 1) // CHUNK, NCH - 1)
        for c in range(waited, hi + 1):
            @pl.when(my == 1)
            def _(c=c):
                _flow_a_relay(k_ref, v_ref, relay_buf,
                              kv_send_sems, relay_recv, c, 0).wait_recv()
                _relay_fwd(relay_buf, kr, vr, fwd_send, recvA, c, 0).start()

            @pl.when(my == 3)
            def _(c=c):
                _flow_a_relay(k_ref, v_ref, relay_buf,
                              kv_send_sems, relay_recv, c, 1).wait_recv()
                _relay_fwd(relay_buf, kr, vr, fwd_send, recvA, c, 1).start()

            @pl.when(my != 0)
            def _(c=c):
                for t in (0, 1):
                    _flow_a_direct(k_ref, v_ref, kr, vr,
                                   kv_send_sems, recvA, 1, c, t).wait_recv()
        waited = max(waited, hi + 1)
        if b == NB - 1:
            @pl.when(my != 1)
            def _():
                for t in (0, 1):
                    _flow_b_rdma(k_ref, v_ref, kr, vr,
                                 fb_send_sems, recvB, 0, 0, t).wait_recv()

        s_b = max(0, QBLK * b - WIN)
        qrows = pl.ds(QBLK * b, QBLK)
        qi = lax.broadcasted_iota(jnp.int32, (QBLK, KBLK), 0) + QBLK * b
        ki = lax.broadcasted_iota(jnp.int32, (QBLK, KBLK), 1) + s_b
        mask = jnp.abs(qi - ki) <= WIN
        acc = jnp.zeros((QBLK, D_MODEL), F32)
        for h in range(HQ):
            q = q_buf[qrows, pl.ds(h * DH, DH)]
            k = kr[pl.ds(s_b, KBLK), h, :]
            s = lax.dot_general(q, k, (((1,), (1,)), ((), ())),
                                preferred_element_type=F32) * SCALE
            s = jnp.where(mask, s, -1e9)
            m = jnp.max(s, axis=1, keepdims=True)
            w = jnp.exp(s - m)
            w = (w / jnp.sum(w, axis=1, keepdims=True)).astype(BF16)
            v = vr[pl.ds(s_b, KBLK), h, :]
            ctx = jnp.dot(w, v, preferred_element_type=F32)
            acc = acc + jnp.dot(ctx.astype(BF16),
                                wo_ref[pl.ds(h * DH, DH), :],
                                preferred_element_type=F32)
        ar_buf[qrows, :] = acc.astype(BF16)

        if b % 2 == 1:
            c = b // 2
            for snd in range(N_DEV):
                if snd != c:
                    @pl.when(my == snd)
                    def _(snd=snd, c=c):
                        pltpu.make_async_remote_copy(
                            src_ref=ar_buf.at[pl.ds(c * CH, CH), :],
                            dst_ref=rs_slots.at[(snd - c - 1) % N_DEV],
                            send_sem=rs_send.at[c],
                            recv_sem=rs_recv.at[(snd - c - 1) % N_DEV],
                            device_id=(c,),
                            device_id_type=MESH,
                        ).start()

    for slot in range(3):
        pltpu.make_async_remote_copy(
            src_ref=ar_buf.at[pl.ds(0, CH), :],
            dst_ref=rs_slots.at[slot],
            send_sem=rs_send.at[0],
            recv_sem=rs_recv.at[slot],
            device_id=(0,),
            device_id_type=MESH,
        ).wait_recv()
    myrows = pl.ds(my * CH, CH)
    red = (ar_buf[myrows, :].astype(F32)
           + rs_slots[0].astype(F32)
           + rs_slots[1].astype(F32)
           + rs_slots[2].astype(F32))
    ar_buf[myrows, :] = red.astype(BF16)

    for off in (1, 2, 3):
        dest = lax.rem(my + off, N_DEV)
        pltpu.make_async_remote_copy(
            src_ref=ar_buf.at[myrows, :],
            dst_ref=ar_buf.at[myrows, :],
            send_sem=ag_send.at[off - 1],
            recv_sem=ag_recv.at[3 - off],
            device_id=(dest,),
            device_id_type=MESH,
        ).start()
    for slot in range(3):
        pltpu.make_async_remote_copy(
            src_ref=ar_buf.at[pl.ds(0, CH), :],
            dst_ref=ar_buf.at[pl.ds(0, CH), :],
            send_sem=ag_send.at[0],
            recv_sem=ag_recv.at[slot],
            device_id=(0,),
            device_id_type=MESH,
        ).wait_recv()

    out_ref[:, :] = ar_buf[:, :].astype(F32)

    @pl.when(my == 0)
    def _():
        for c in range(NCH):
            for t in (0, 1):
                _flow_a_direct(k_ref, v_ref, kr, vr,
                               kv_send_sems, recvA, 1, c, t).wait_send()
                _flow_a_direct(k_ref, v_ref, kr, vr,
                               kv_send_sems, recvA, 3, c, t).wait_send()
                _flow_a_relay(k_ref, v_ref, relay_buf,
                              kv_send_sems, relay_recv, c, t).wait_send()

    @pl.when(my == 1)
    def _():
        for i, j in enumerate((0, 2, 3)):
            for t in (0, 1):
                _flow_b_rdma(k_ref, v_ref, kr, vr,
                             fb_send_sems, recvB, i, j, t).wait_send()
        for c in range(NCH):
            _relay_fwd(relay_buf, kr, vr, fwd_send, recvA, c, 0).wait_send()

    @pl.when(my == 3)
    def _():
        for c in range(NCH):
            _relay_fwd(relay_buf, kr, vr, fwd_send, recvA, c, 1).wait_send()

    for snd in range(N_DEV):
        @pl.when(my == snd)
        def _(snd=snd):
            for c in range(N_DEV):
                if c != snd:
                    pltpu.make_async_remote_copy(
                        src_ref=ar_buf.at[pl.ds(0, CH), :],
                        dst_ref=rs_slots.at[0],
                        send_sem=rs_send.at[c],
                        recv_sem=rs_recv.at[0],
                        device_id=(0,),
                        device_id_type=MESH,
                    ).wait_send()

    for off in (1, 2, 3):
        pltpu.make_async_remote_copy(
            src_ref=ar_buf.at[pl.ds(0, CH), :],
            dst_ref=rs_slots.at[0],
            send_sem=ag_send.at[off - 1],
            recv_sem=rs_recv.at[0],
            device_id=(0,),
            device_id_type=MESH,
        ).wait_send()


def kernel(x, Wq, K_ext, V_ext, Wo):
    x2 = x.reshape(SQ, D_MODEL).astype(BF16)
    Wqb = Wq.astype(BF16)
    Wob = Wo.astype(BF16)
    Kb = K_ext.reshape(KV_LOC, 4 * HQ, DH).astype(BF16)
    Vb = V_ext.reshape(KV_LOC, 4 * HQ, DH).astype(BF16)

    out2 = pl.pallas_call(
        _body,
        out_shape=jax.ShapeDtypeStruct((SQ, D_MODEL), F32),
        in_specs=[
            pl.BlockSpec(memory_space=pltpu.VMEM),
            pl.BlockSpec(memory_space=pltpu.VMEM),
            pl.BlockSpec(memory_space=pl.ANY),
            pl.BlockSpec(memory_space=pl.ANY),
            pl.BlockSpec(memory_space=pltpu.VMEM),
        ],
        out_specs=pl.BlockSpec(memory_space=pltpu.VMEM),
        scratch_shapes=[
            pltpu.VMEM((SQ, D_MODEL), BF16),
            pltpu.VMEM((KV_NEED, HQ, DH), BF16),
            pltpu.VMEM((KV_NEED, HQ, DH), BF16),
            pltpu.VMEM((SQ, D_MODEL), BF16),
            pltpu.VMEM((3, CH, D_MODEL), BF16),
            pltpu.VMEM((KV_LOC, HQ, DH), BF16),
            pltpu.SemaphoreType.DMA((24,)),
            pltpu.SemaphoreType.DMA((6,)),
            pltpu.SemaphoreType.DMA((2,)),
            pltpu.SemaphoreType.DMA((NCH,)),
            pltpu.SemaphoreType.DMA((NCH,)),
            pltpu.SemaphoreType.DMA((2 * NCH,)),
            pltpu.SemaphoreType.DMA((2,)),
            pltpu.SemaphoreType.DMA((N_DEV,)),
            pltpu.SemaphoreType.DMA((3,)),
            pltpu.SemaphoreType.DMA((3,)),
            pltpu.SemaphoreType.DMA((3,)),
        ],
        compiler_params=pltpu.CompilerParams(
            collective_id=0,
            vmem_limit_bytes=128 * 1024 * 1024,
        ),
    )(x2, Wqb, Kb, Vb, Wob)
    return out2.reshape(1, SQ, D_MODEL)
